# Initial kernel scaffold; baseline (speedup 1.0000x reference)
import jax
import jax.numpy as jnp
from jax.experimental import pallas as pl


def kernel(image_h, image_w, features, proposals, W1, b1, W2, b2, Wc, bc, Wr, br):
    raise NotImplementedError("write your pallas kernel here")


# speedup vs baseline: 5.8735x; 5.8735x over previous
import functools
import numpy as np
import jax
import jax.numpy as jnp
from jax import lax
from jax.experimental import pallas as pl
from jax.experimental.pallas import tpu as pltpu
from jax.experimental.pallas import tpu_sc as plsc

_N = 1000
_C = 256
_FH = 50
_FW = 50
_NCLS = 80
_OUT = 7
_HID = 1024
_K = _C * _OUT * _OUT
_SCORE_THRESH = 0.05
_NMS_THRESH = 0.5
_DETS = 100
_SCALE = 1.0 / 16.0
_CLIPV = float(np.log(1000.0 / 16.0))

_NPAD = 1024
_NPTS = _NPAD * _OUT * _OUT
_NW = 32
_BPW = _NPTS // _NW
_GCH = 112
_NCHUNK = _BPW // _GCH
_R = 625
_L = 128


def _prep_kernel(prop_ref, i00_ref, i01_ref, i10_ref, i11_ref,
                 lx_ref, ly_ref):
    props = prop_ref[...]
    b = props * _SCALE
    x1 = b[:, 0:1]
    y1 = b[:, 1:2]
    x2 = b[:, 2:3]
    y2 = b[:, 3:4]
    bw = jnp.maximum(x2 - x1, 1e-3)
    bh = jnp.maximum(y2 - y1, 1e-3)
    g = (lax.broadcasted_iota(jnp.int32, (_NPAD, _OUT), 1)
         .astype(jnp.float32) + 0.5) / _OUT
    px = (x1 + g * bw) - 0.5
    py = (y1 + g * bh) - 0.5
    x0f = jnp.floor(px)
    y0f = jnp.floor(py)
    lx = px - x0f
    ly = py - y0f
    x0i = jnp.clip(x0f.astype(jnp.int32), 0, _FW - 1)
    x1i = jnp.clip(x0i + 1, 0, _FW - 1)
    y0i = jnp.clip(y0f.astype(jnp.int32), 0, _FH - 1)
    y1i = jnp.clip(y0i + 1, 0, _FH - 1)

    def grid49(yv, xv):
        return yv[:, :, None] * _FW + xv[:, None, :]

    i00_ref[...] = grid49(y0i, x0i)
    i01_ref[...] = grid49(y0i, x1i)
    i10_ref[...] = grid49(y1i, x0i)
    i11_ref[...] = grid49(y1i, x1i)
    lx_ref[...] = jnp.broadcast_to(lx[:, None, :], (_NPAD, _OUT, _OUT))
    ly_ref[...] = jnp.broadcast_to(ly[:, :, None], (_NPAD, _OUT, _OUT))


def _sc_gather_body(table_hbm, i00, i01, i10, i11,
                    o00, o01, o10, o11, idx_v, rows_v, sem):
    wid = lax.axis_index("s") * 2 + lax.axis_index("c")
    base = wid * _BPW
    for idx_hbm, out_hbm in ((i00, o00), (i01, o01), (i10, o10), (i11, o11)):
        def chunk(c, _):
            off = base + c * _GCH
            pltpu.sync_copy(idx_hbm.at[pl.ds(off, _GCH)], idx_v)
            pltpu.async_copy(table_hbm.at[idx_v], rows_v, sem).wait()
            pltpu.sync_copy(rows_v, out_hbm.at[pl.ds(off, _GCH)])
            return 0
        lax.fori_loop(0, _NCHUNK, chunk, 0)


def _sc_gather(table, i00, i01, i10, i11):
    mesh = plsc.VectorSubcoreMesh(core_axis_name="c", subcore_axis_name="s")
    run = functools.partial(
        pl.kernel, mesh=mesh,
        out_type=[jax.ShapeDtypeStruct((_NPTS, _C), jnp.float32)] * 4,
        scratch_types=[
            pltpu.VMEM((_GCH,), jnp.int32),
            pltpu.VMEM((_GCH, _C), jnp.float32),
            pltpu.SemaphoreType.DMA,
        ],
    )(_sc_gather_body)
    return run(table, i00, i01, i10, i11)


_LB = 32


def _lerp_kernel(g00_ref, g01_ref, g10_ref, g11_ref, lx_ref, ly_ref,
                 val_ref):
    lx = lx_ref[...]
    ly = ly_ref[...]
    wx1 = 1.0 - lx
    wy1 = 1.0 - ly
    val_ref[...] = ((g00_ref[...] * wx1) * wy1
                    + (g01_ref[...] * lx) * wy1
                    + (g10_ref[...] * wx1) * ly
                    + (g11_ref[...] * lx) * ly)


def _mlp1_kernel(x_ref, w1_ref, b1_ref, h1_ref):
    h1_ref[...] = jax.nn.relu(
        jnp.dot(x_ref[...], w1_ref[...], preferred_element_type=jnp.float32)
        + b1_ref[...])


def _mlp2_kernel(cons_ref, h1_ref, w2_ref, b2_ref, wc_ref, bc_ref,
                 wr_ref, brv_ref, prop_ref,
                 sc_out, msk_out, bx1_out, by1_out, bx2_out, by2_out):
    h2 = jax.nn.relu(
        jnp.dot(h1_ref[...], w2_ref[...], preferred_element_type=jnp.float32)
        + b2_ref[...])
    cls = jnp.dot(h2, wc_ref[...],
                  preferred_element_type=jnp.float32) + bc_ref[...]
    dx = jnp.dot(h2, wr_ref[0 * _HID:1 * _HID],
                 preferred_element_type=jnp.float32) + brv_ref[0:1, :]
    dy = jnp.dot(h2, wr_ref[1 * _HID:2 * _HID],
                 preferred_element_type=jnp.float32) + brv_ref[1:2, :]
    dw = jnp.dot(h2, wr_ref[2 * _HID:3 * _HID],
                 preferred_element_type=jnp.float32) + brv_ref[2:3, :]
    dh = jnp.dot(h2, wr_ref[3 * _HID:4 * _HID],
                 preferred_element_type=jnp.float32) + brv_ref[3:4, :]

    p = prop_ref[...]
    widths = p[:, 2:3] - p[:, 0:1]
    heights = p[:, 3:4] - p[:, 1:2]
    ctr_x = p[:, 0:1] + 0.5 * widths
    ctr_y = p[:, 1:2] + 0.5 * heights
    dx = dx / 10.0
    dy = dy / 10.0
    dw = jnp.minimum(dw / 5.0, _CLIPV)
    dh = jnp.minimum(dh / 5.0, _CLIPV)
    pcx = dx * widths + ctr_x
    pcy = dy * heights + ctr_y
    pw = jnp.exp(dw) * widths
    ph = jnp.exp(dh) * heights
    iw = cons_ref[0]
    ih = cons_ref[1]
    x1 = jnp.clip(pcx - 0.5 * pw, 0.0, iw)
    y1 = jnp.clip(pcy - 0.5 * ph, 0.0, ih)
    x2 = jnp.clip(pcx + 0.5 * pw, 0.0, iw)
    y2 = jnp.clip(pcy + 0.5 * ph, 0.0, ih)
    scores = jax.nn.sigmoid(cls)
    ws = x2 - x1
    hs = y2 - y1
    rowid = lax.broadcasted_iota(jnp.int32, (_NPAD, 1), 0)
    valid = ((scores > _SCORE_THRESH) & (ws >= 0.01) & (hs >= 0.01)
             & (rowid < _N))
    sc_out[...] = scores
    msk_out[...] = jnp.where(valid, scores, -1e9)
    bx1_out[...] = x1
    by1_out[...] = y1
    bx2_out[...] = x2
    by2_out[...] = y2


def _nms_kernel(cons_ref, msk_ref, sc_ref, bx1_ref, by1_ref, bx2_ref,
                by2_ref, cls_ref,
                bx1o, by1o, bx2o, by2o, sco, lbo, run_ref):
    off_unit = jnp.maximum(cons_ref[0], cons_ref[1]) + 2.0
    clsf = cls_ref[...].astype(jnp.float32)
    offs = clsf * off_unit
    b1 = bx1_ref[...]
    b2 = by1_ref[...]
    b3 = bx2_ref[...]
    b4 = by2_ref[...]
    ox1 = b1 + offs
    oy1 = b2 + offs
    ox2 = b3 + offs
    oy2 = b4 + offs
    areas = (ox2 - ox1) * (oy2 - oy1)
    flat = (lax.broadcasted_iota(jnp.int32, (_R, _L), 0) * _L
            + lax.broadcasted_iota(jnp.int32, (_R, _L), 1))
    run_ref[...] = msk_ref[...]
    lane = lax.broadcasted_iota(jnp.int32, (1, _L), 1)
    z = jnp.zeros((1, _L), jnp.float32)
    zl = jnp.zeros((1, _L), jnp.int32)

    def body(t, carry):
        vb1, vy1, vb2, vy2, vsc, vlb = carry
        a = run_ref[...]
        m = jnp.max(a)
        idx = jnp.min(jnp.where(a == m, flat, jnp.int32(2147483647)))
        onehot = flat == idx

        def pickf(arr):
            return jnp.sum(jnp.where(onehot, arr, 0.0))

        px1 = pickf(ox1)
        py1 = pickf(oy1)
        px2 = pickf(ox2)
        py2 = pickf(oy2)
        parea = (px2 - px1) * (py2 - py1)
        morig = pickf(msk_ref[...])
        psc = pickf(sc_ref[...])
        pb1 = pickf(b1)
        pb2 = pickf(b2)
        pb3 = pickf(b3)
        pb4 = pickf(b4)
        pcls = jnp.sum(jnp.where(onehot, cls_ref[...], 0))

        xx1 = jnp.maximum(px1, ox1)
        yy1 = jnp.maximum(py1, oy1)
        xx2 = jnp.minimum(px2, ox2)
        yy2 = jnp.minimum(py2, oy2)
        inter = jnp.maximum(xx2 - xx1, 0.0) * jnp.maximum(yy2 - yy1, 0.0)
        iou = inter / (parea + areas - inter + 1e-9)
        run_ref[...] = jnp.where((iou > _NMS_THRESH) | onehot, -1e9, a)

        kv = morig > -1e8
        sel = lane == t
        vb1 = jnp.where(sel, jnp.where(kv, pb1, 0.0), vb1)
        vy1 = jnp.where(sel, jnp.where(kv, pb2, 0.0), vy1)
        vb2 = jnp.where(sel, jnp.where(kv, pb3, 0.0), vb2)
        vy2 = jnp.where(sel, jnp.where(kv, pb4, 0.0), vy2)
        vsc = jnp.where(sel, jnp.where(kv, psc, 0.0), vsc)
        vlb = jnp.where(sel, jnp.where(kv, pcls, -1), vlb)
        return (vb1, vy1, vb2, vy2, vsc, vlb)

    out = lax.fori_loop(0, _DETS, body, (z, z, z, z, z, zl))
    bx1o[...] = out[0]
    by1o[...] = out[1]
    bx2o[...] = out[2]
    by2o[...] = out[3]
    sco[...] = out[4]
    lbo[...] = out[5]


def kernel(image_h, image_w, features, proposals,
           W1, b1, W2, b2, Wc, bc, Wr, br):
    f32 = jnp.float32
    featT = features[0].transpose(1, 2, 0).reshape(_FH * _FW, _C)
    padbox = jnp.broadcast_to(
        jnp.asarray([[0.0, 0.0, 160.0, 160.0]], f32), (_NPAD - _N, 4))
    props_p = jnp.concatenate([proposals.astype(f32), padbox], axis=0)

    i00, i01, i10, i11, lx3, ly3 = pl.pallas_call(
        _prep_kernel,
        in_specs=[pl.BlockSpec((_NPAD, 4), lambda: (0, 0))],
        out_specs=[pl.BlockSpec((_NPAD, _OUT, _OUT), lambda: (0, 0, 0))] * 6,
        out_shape=([jax.ShapeDtypeStruct((_NPAD, _OUT, _OUT), jnp.int32)] * 4
                   + [jax.ShapeDtypeStruct((_NPAD, _OUT, _OUT), f32)] * 2),
    )(props_p)
    lx49 = lx3.reshape(_NPTS, 1)
    ly49 = ly3.reshape(_NPTS, 1)

    g00, g01, g10, g11 = _sc_gather(
        featT, i00.reshape(_NPTS), i01.reshape(_NPTS),
        i10.reshape(_NPTS), i11.reshape(_NPTS))

    val = pl.pallas_call(
        _lerp_kernel,
        grid=(_NPAD // _LB,),
        in_specs=(
            [pl.BlockSpec((_LB * _OUT * _OUT, _C), lambda m: (m, 0))] * 4
            + [pl.BlockSpec((_LB * _OUT * _OUT, 1), lambda m: (m, 0))] * 2),
        out_specs=pl.BlockSpec((_LB * _OUT * _OUT, _C), lambda m: (m, 0)),
        out_shape=jax.ShapeDtypeStruct((_NPTS, _C), f32),
    )(g00, g01, g10, g11, lx49, ly49)

    x2d = val.reshape(_NPAD, _OUT * _OUT, _C).transpose(0, 2, 1) \
        .reshape(_NPAD, _K)

    h1 = pl.pallas_call(
        _mlp1_kernel,
        grid=(4, 8),
        in_specs=[
            pl.BlockSpec((128, _K), lambda n, m: (m, 0)),
            pl.BlockSpec((_K, 256), lambda n, m: (0, n)),
            pl.BlockSpec((1, 256), lambda n, m: (0, n)),
        ],
        out_specs=pl.BlockSpec((128, 256), lambda n, m: (m, n)),
        out_shape=jax.ShapeDtypeStruct((_NPAD, _HID), f32),
    )(x2d, W1, b1.reshape(1, _HID))

    Wr4 = Wr.reshape(_HID, _NCLS, 4)
    Wrs = jnp.concatenate([Wr4[:, :, 0], Wr4[:, :, 1],
                           Wr4[:, :, 2], Wr4[:, :, 3]], axis=0)
    brv = br.reshape(_NCLS, 4).T
    iw = jnp.asarray(image_w).astype(f32)
    ih = jnp.asarray(image_h).astype(f32)
    cons = jnp.stack([iw, ih]).reshape(2)

    scores, msk, bx1, by1, bx2, by2 = pl.pallas_call(
        _mlp2_kernel,
        in_specs=[
            pl.BlockSpec(memory_space=pltpu.SMEM),
            pl.BlockSpec((_NPAD, _HID), lambda: (0, 0)),
            pl.BlockSpec((_HID, _HID), lambda: (0, 0)),
            pl.BlockSpec((1, _HID), lambda: (0, 0)),
            pl.BlockSpec((_HID, _NCLS), lambda: (0, 0)),
            pl.BlockSpec((1, _NCLS), lambda: (0, 0)),
            pl.BlockSpec((4 * _HID, _NCLS), lambda: (0, 0)),
            pl.BlockSpec((4, _NCLS), lambda: (0, 0)),
            pl.BlockSpec((_NPAD, 4), lambda: (0, 0)),
        ],
        out_specs=[pl.BlockSpec((_NPAD, _NCLS), lambda: (0, 0))] * 6,
        out_shape=[jax.ShapeDtypeStruct((_NPAD, _NCLS), f32)] * 6,
    )(cons, h1, W2, b2.reshape(1, _HID), Wc, bc.reshape(1, _NCLS),
      Wrs, brv, props_p)

    cls2 = jnp.broadcast_to(jnp.arange(_NCLS, dtype=jnp.int32),
                            (_N, _NCLS)).reshape(_R, _L)
    rs = lambda a: a[:_N].reshape(_R, _L)

    nms_outs = pl.pallas_call(
        _nms_kernel,
        in_specs=([pl.BlockSpec(memory_space=pltpu.SMEM)]
                  + [pl.BlockSpec((_R, _L), lambda: (0, 0))] * 7),
        out_specs=[pl.BlockSpec((1, _L), lambda: (0, 0))] * 6,
        out_shape=([jax.ShapeDtypeStruct((1, _L), f32)] * 5
                   + [jax.ShapeDtypeStruct((1, _L), jnp.int32)]),
        scratch_shapes=[pltpu.VMEM((_R, _L), f32)],
    )(cons, rs(msk), rs(scores), rs(bx1), rs(by1), rs(bx2), rs(by2), cls2)

    vb1, vy1, vb2, vy2, vsc, vlb = nms_outs
    boxes_out = jnp.stack([vb1[0, :_DETS], vy1[0, :_DETS],
                           vb2[0, :_DETS], vy2[0, :_DETS]], axis=1)
    scores_out = vsc[0, :_DETS]
    labels_out = vlb[0, :_DETS]
    return boxes_out, scores_out, labels_out
